# baseline (speedup 1.0000x reference)
import functools

import jax
import jax.numpy as jnp
from jax import lax
from jax.experimental import pallas as pl
from jax.experimental.pallas import tpu as pltpu
from jax.experimental.pallas import tpu_sc as plsc

N = 10000
E = 320000
D = 128
H = 16

NC = 2
NS = 16
NW = NC * NS
BLK = 128
G = 8
SG = G * BLK
TCH = 320
EPAD = TCH * SG
K0 = 6
K1 = 20 - K0
RPT = 632
NPAD = NS * RPT


def _seg_sum_sc(table, edgb, zrow, with_gather):
    mesh = plsc.VectorSubcoreMesh(core_axis_name="c", subcore_axis_name="s")

    @functools.partial(
        pl.kernel,
        out_type=jax.ShapeDtypeStruct((NC, NPAD, H), jnp.float32),
        mesh=mesh,
        scratch_types=[
            pltpu.VMEM_SHARED((NPAD, H), jnp.float32),
            pltpu.VMEM((2, G, BLK), jnp.int32),
            pltpu.VMEM((2, G, BLK), jnp.int32),
            pltpu.VMEM((SG, H), jnp.float32),
            pltpu.VMEM((SG, H), jnp.float32),
            pltpu.VMEM((RPT, H), jnp.float32),
            pltpu.SemaphoreType.DMA,
            pltpu.SemaphoreType.DMA,
            pltpu.SemaphoreType.DMA,
            pltpu.SemaphoreType.DMA,
        ],
        compiler_params=pltpu.CompilerParams(use_tc_tiling_on_sc=False),
    )
    def k(table_h, edg_h, zrow_h, out_h,
          acc, eA, eB, rA, rB, tbuf, gsA, gsB, ssA, ssB):
        c = lax.axis_index("c")
        s = lax.axis_index("s")
        base = jnp.where(c == 0, s * K0, NS * K0 + s * K1)
        npair = jnp.where(c == 0, K0 // 2, K1 // 2)

        def stage(ebuf, sup):
            pltpu.sync_copy(edg_h.at[base + sup], ebuf)

        def fire_g(ebuf, rbuf, sem):
            for j in range(G):
                pltpu.async_copy(table_h.at[ebuf.at[0, j]],
                                 rbuf.at[pl.ds(j * BLK, BLK)], sem)

        def drain_g(rbuf, sem):
            pltpu.make_async_copy(table_h.at[pl.ds(0, SG)], rbuf, sem).wait()

        def fire_s(ebuf, rbuf, sem):
            for j in range(G):
                pltpu.async_copy(rbuf.at[pl.ds(j * BLK, BLK)],
                                 acc.at[ebuf.at[1, j]], sem, add=True)

        def drain_s(rbuf, sem):
            pltpu.make_async_copy(rbuf, acc.at[pl.ds(0, SG)], sem).wait()

        stage(eA, 0)
        if with_gather:
            fire_g(eA, rA, gsA)
        else:
            pltpu.sync_copy(table_h.at[pl.ds(0, SG)], rA)
        pltpu.sync_copy(zrow_h, tbuf)
        pltpu.sync_copy(tbuf, acc.at[pl.ds(s * RPT, RPT)])
        plsc.subcore_barrier()
        if not with_gather:
            fire_s(eA, rA, ssA)

        def body(i, carry):
            i2 = 2 * i
            if with_gather:
                stage(eB, i2 + 1)
                fire_g(eB, rB, gsB)
                drain_g(rA, gsA)
                fire_s(eA, rA, ssA)
                drain_s(rA, ssA)

                @pl.when(i < npair - 1)
                def _():
                    stage(eA, i2 + 2)
                    fire_g(eA, rA, gsA)

                drain_g(rB, gsB)
                fire_s(eB, rB, ssB)
                drain_s(rB, ssB)
            else:
                stage(eB, i2 + 1)
                fire_s(eB, rA, ssB)
                drain_s(rA, ssA)

                @pl.when(i < npair - 1)
                def _():
                    stage(eA, i2 + 2)
                    fire_s(eA, rA, ssA)

                drain_s(rA, ssB)
            return carry

        lax.fori_loop(0, npair, body, 0)
        plsc.subcore_barrier()

        pltpu.sync_copy(acc.at[pl.ds(s * RPT, RPT)], tbuf)
        pltpu.sync_copy(tbuf, out_h.at[c, pl.ds(s * RPT, RPT)])

    return k(table, edgb, zrow)


def _bn_gelu(y, g, be):
    m = jnp.mean(y, axis=0, keepdims=True)
    v = jnp.mean((y - m) * (y - m), axis=0, keepdims=True)
    return jax.nn.gelu((y - m) / jnp.sqrt(v + 1e-5) * g + be)


def _tc_pre(x, W1, degp):
    def k(x_r, w_r, dg_r, dinv_r, h1p_r):
        deg = dg_r[0, :N, :] + dg_r[1, :N, :] + 1.0
        dinv = lax.rsqrt(deg)
        h1 = jnp.dot(x_r[...], w_r[...], preferred_element_type=jnp.float32, precision=lax.Precision.HIGHEST)
        dinv_r[...] = dinv
        h1p_r[...] = h1 * dinv

    return pl.pallas_call(
        k,
        out_shape=(jax.ShapeDtypeStruct((N, H), jnp.float32),
                   jax.ShapeDtypeStruct((N, H), jnp.float32)),
    )(x, W1, degp)


def _tc_mid(sp, hp, dinv, b, g, be, W):
    def k(sp_r, hp_r, dinv_r, b_r, g_r, be_r, *rest):
        w_r, out_r = (rest[0], rest[1]) if len(rest) == 2 else (None, rest[0])
        y = dinv_r[...] * (sp_r[0, :N, :] + sp_r[1, :N, :] + hp_r[...]) + b_r[...]
        z = _bn_gelu(y, g_r[...], be_r[...])
        if w_r is not None:
            z = jnp.dot(z, w_r[...], preferred_element_type=jnp.float32, precision=lax.Precision.HIGHEST)
        out_r[...] = z * dinv_r[...]

    args = [sp, hp, dinv, b.reshape(1, H), g.reshape(1, H), be.reshape(1, H)]
    if W is not None:
        args.append(W)
    return pl.pallas_call(
        k,
        out_shape=jax.ShapeDtypeStruct((N, H), jnp.float32),
    )(*args)


def _tc_post(s3, z2p, dinv, W3, b3, g3, be3, x,
             fw1, fb1, fw2, fb2, fw3, fb3, fw4, fb4):
    def k(s3_r, z2p_r, dinv_r, w3_r, b3_r, g3_r, be3_r, x_r,
          f1_r, c1_r, f2_r, c2_r, f3_r, c3_r, f4_r, c4_r, out_r):
        t = dinv_r[...] * (s3_r[0, :N, :] + s3_r[1, :N, :] + z2p_r[...])
        h3 = jnp.dot(t, w3_r[...], preferred_element_type=jnp.float32, precision=lax.Precision.HIGHEST) + b3_r[...]
        z3 = _bn_gelu(h3, g3_r[...], be3_r[...])
        hh = x_r[...] + z3
        v = jnp.mean(hh, axis=0, keepdims=True)
        v = jax.nn.gelu(jnp.dot(v, f1_r[...], preferred_element_type=jnp.float32, precision=lax.Precision.HIGHEST) + c1_r[...])
        v = jax.nn.gelu(jnp.dot(v, f2_r[...], preferred_element_type=jnp.float32, precision=lax.Precision.HIGHEST) + c2_r[...])
        v = jax.nn.gelu(jnp.dot(v, f3_r[...], preferred_element_type=jnp.float32, precision=lax.Precision.HIGHEST) + c3_r[...])
        out_r[...] = jnp.dot(v, f4_r[...], preferred_element_type=jnp.float32, precision=lax.Precision.HIGHEST) + c4_r[...]

    return pl.pallas_call(
        k,
        out_shape=jax.ShapeDtypeStruct((1, 1), jnp.float32),
    )(s3, z2p, dinv, W3, b3.reshape(1, D), g3.reshape(1, D), be3.reshape(1, D),
      x, fw1, fb1.reshape(1, 128), fw2, fb2.reshape(1, 64),
      fw3, fb3.reshape(1, 32), fw4, fb4.reshape(1, 1))


def kernel(x, edge_index, W1, b1, g1, be1, W2, b2, g2, be2, W3, b3, g3, be3,
           fw1, fb1, fw2, fb2, fw3, fb3, fw4, fb4):
    ei = edge_index.astype(jnp.int32)
    pad = EPAD - E
    srcb = jnp.concatenate([ei[0], jnp.zeros((pad,), jnp.int32)])
    srcb = srcb.reshape(TCH, G, BLK)
    dstb = jnp.concatenate([ei[1], jnp.full((pad,), N, jnp.int32)])
    dstb = dstb.reshape(TCH, G, BLK)
    edgb = jnp.stack([srcb, dstb], axis=1)
    ones_t = jnp.ones((N, H), jnp.float32)
    zrow = jnp.zeros((RPT, H), jnp.float32)

    degp = _seg_sum_sc(ones_t, edgb, zrow, with_gather=False)
    dinv, h1p = _tc_pre(x, W1, degp)

    s1 = _seg_sum_sc(h1p, edgb, zrow, with_gather=True)
    h2p = _tc_mid(s1, h1p, dinv, b1, g1, be1, W2)

    s2 = _seg_sum_sc(h2p, edgb, zrow, with_gather=True)
    z2p = _tc_mid(s2, h2p, dinv, b2, g2, be2, None)

    s3 = _seg_sum_sc(z2p, edgb, zrow, with_gather=True)
    out = _tc_post(s3, z2p, dinv, W3, b3, g3, be3, x,
                   fw1, fb1, fw2, fb2, fw3, fb3, fw4, fb4)
    return out.reshape((1,))


# speedup vs baseline: 1.1154x; 1.1154x over previous
import functools

import jax
import jax.numpy as jnp
from jax import lax
from jax.experimental import pallas as pl
from jax.experimental.pallas import tpu as pltpu
from jax.experimental.pallas import tpu_sc as plsc

N = 10000
E = 320000
D = 128
H = 16

NC = 2
NS = 16
NW = NC * NS
BLK = 128
G = 8
SG = G * BLK
TCH = 320
EPAD = TCH * SG
K0 = 14
K1 = 20 - K0
RPT = 632
NPAD = NS * RPT


def _seg_sum_sc(table, edgb, zrow, with_gather):
    mesh = plsc.VectorSubcoreMesh(core_axis_name="c", subcore_axis_name="s")

    @functools.partial(
        pl.kernel,
        out_type=jax.ShapeDtypeStruct((NC, NPAD, H), jnp.float32),
        mesh=mesh,
        scratch_types=[
            pltpu.VMEM_SHARED((NPAD, H), jnp.float32),
            pltpu.VMEM((2, G, BLK), jnp.int32),
            pltpu.VMEM((2, G, BLK), jnp.int32),
            pltpu.VMEM((SG, H), jnp.float32),
            pltpu.VMEM((SG, H), jnp.float32),
            pltpu.VMEM((RPT, H), jnp.float32),
            pltpu.SemaphoreType.DMA,
            pltpu.SemaphoreType.DMA,
            pltpu.SemaphoreType.DMA,
            pltpu.SemaphoreType.DMA,
        ],
        compiler_params=pltpu.CompilerParams(use_tc_tiling_on_sc=False),
    )
    def k(table_h, edg_h, zrow_h, out_h,
          acc, eA, eB, rA, rB, tbuf, gsA, gsB, ssA, ssB):
        c = lax.axis_index("c")
        s = lax.axis_index("s")
        base = jnp.where(c == 0, s * K0, NS * K0 + s * K1)
        npair = jnp.where(c == 0, K0 // 2, K1 // 2)

        def stage(ebuf, sup):
            pltpu.sync_copy(edg_h.at[base + sup], ebuf)

        def fire_g(ebuf, rbuf, sem):
            for j in range(G):
                pltpu.async_copy(table_h.at[ebuf.at[0, j]],
                                 rbuf.at[pl.ds(j * BLK, BLK)], sem)

        def drain_g(rbuf, sem):
            pltpu.make_async_copy(table_h.at[pl.ds(0, SG)], rbuf, sem).wait()

        def fire_s(ebuf, rbuf, sem):
            for j in range(G):
                pltpu.async_copy(rbuf.at[pl.ds(j * BLK, BLK)],
                                 acc.at[ebuf.at[1, j]], sem, add=True)

        def drain_s(rbuf, sem):
            pltpu.make_async_copy(rbuf, acc.at[pl.ds(0, SG)], sem).wait()

        stage(eA, 0)
        if with_gather:
            fire_g(eA, rA, gsA)
        else:
            pltpu.sync_copy(table_h.at[pl.ds(0, SG)], rA)
        pltpu.sync_copy(zrow_h, tbuf)
        pltpu.sync_copy(tbuf, acc.at[pl.ds(s * RPT, RPT)])
        plsc.subcore_barrier()
        if not with_gather:
            fire_s(eA, rA, ssA)

        def body(i, carry):
            i2 = 2 * i
            if with_gather:
                stage(eB, i2 + 1)
                fire_g(eB, rB, gsB)
                drain_g(rA, gsA)
                fire_s(eA, rA, ssA)
                drain_s(rA, ssA)

                @pl.when(i < npair - 1)
                def _():
                    stage(eA, i2 + 2)
                    fire_g(eA, rA, gsA)

                drain_g(rB, gsB)
                fire_s(eB, rB, ssB)
                drain_s(rB, ssB)
            else:
                stage(eB, i2 + 1)
                fire_s(eB, rA, ssB)
                drain_s(rA, ssA)

                @pl.when(i < npair - 1)
                def _():
                    stage(eA, i2 + 2)
                    fire_s(eA, rA, ssA)

                drain_s(rA, ssB)
            return carry

        lax.fori_loop(0, npair, body, 0)
        plsc.subcore_barrier()

        pltpu.sync_copy(acc.at[pl.ds(s * RPT, RPT)], tbuf)
        pltpu.sync_copy(tbuf, out_h.at[c, pl.ds(s * RPT, RPT)])

    return k(table, edgb, zrow)


def _bn_gelu(y, g, be):
    m = jnp.mean(y, axis=0, keepdims=True)
    v = jnp.mean((y - m) * (y - m), axis=0, keepdims=True)
    return jax.nn.gelu((y - m) / jnp.sqrt(v + 1e-5) * g + be)


def _tc_pre(x, W1, degp):
    def k(x_r, w_r, dg_r, dinv_r, h1p_r):
        deg = dg_r[0, :N, :] + dg_r[1, :N, :] + 1.0
        dinv = lax.rsqrt(deg)
        h1 = jnp.dot(x_r[...], w_r[...], preferred_element_type=jnp.float32, precision=lax.Precision.HIGHEST)
        dinv_r[...] = dinv
        h1p_r[...] = h1 * dinv

    return pl.pallas_call(
        k,
        out_shape=(jax.ShapeDtypeStruct((N, H), jnp.float32),
                   jax.ShapeDtypeStruct((N, H), jnp.float32)),
    )(x, W1, degp)


def _tc_mid(sp, hp, dinv, b, g, be, W):
    def k(sp_r, hp_r, dinv_r, b_r, g_r, be_r, *rest):
        w_r, out_r = (rest[0], rest[1]) if len(rest) == 2 else (None, rest[0])
        y = dinv_r[...] * (sp_r[0, :N, :] + sp_r[1, :N, :] + hp_r[...]) + b_r[...]
        z = _bn_gelu(y, g_r[...], be_r[...])
        if w_r is not None:
            z = jnp.dot(z, w_r[...], preferred_element_type=jnp.float32, precision=lax.Precision.HIGHEST)
        out_r[...] = z * dinv_r[...]

    args = [sp, hp, dinv, b.reshape(1, H), g.reshape(1, H), be.reshape(1, H)]
    if W is not None:
        args.append(W)
    return pl.pallas_call(
        k,
        out_shape=jax.ShapeDtypeStruct((N, H), jnp.float32),
    )(*args)


def _tc_post(s3, z2p, dinv, W3, b3, g3, be3, x,
             fw1, fb1, fw2, fb2, fw3, fb3, fw4, fb4):
    def k(s3_r, z2p_r, dinv_r, w3_r, b3_r, g3_r, be3_r, x_r,
          f1_r, c1_r, f2_r, c2_r, f3_r, c3_r, f4_r, c4_r, out_r):
        t = dinv_r[...] * (s3_r[0, :N, :] + s3_r[1, :N, :] + z2p_r[...])
        h3 = jnp.dot(t, w3_r[...], preferred_element_type=jnp.float32, precision=lax.Precision.HIGHEST) + b3_r[...]
        z3 = _bn_gelu(h3, g3_r[...], be3_r[...])
        hh = x_r[...] + z3
        v = jnp.mean(hh, axis=0, keepdims=True)
        v = jax.nn.gelu(jnp.dot(v, f1_r[...], preferred_element_type=jnp.float32, precision=lax.Precision.HIGHEST) + c1_r[...])
        v = jax.nn.gelu(jnp.dot(v, f2_r[...], preferred_element_type=jnp.float32, precision=lax.Precision.HIGHEST) + c2_r[...])
        v = jax.nn.gelu(jnp.dot(v, f3_r[...], preferred_element_type=jnp.float32, precision=lax.Precision.HIGHEST) + c3_r[...])
        out_r[...] = jnp.dot(v, f4_r[...], preferred_element_type=jnp.float32, precision=lax.Precision.HIGHEST) + c4_r[...]

    return pl.pallas_call(
        k,
        out_shape=jax.ShapeDtypeStruct((1, 1), jnp.float32),
    )(s3, z2p, dinv, W3, b3.reshape(1, D), g3.reshape(1, D), be3.reshape(1, D),
      x, fw1, fb1.reshape(1, 128), fw2, fb2.reshape(1, 64),
      fw3, fb3.reshape(1, 32), fw4, fb4.reshape(1, 1))


def kernel(x, edge_index, W1, b1, g1, be1, W2, b2, g2, be2, W3, b3, g3, be3,
           fw1, fb1, fw2, fb2, fw3, fb3, fw4, fb4):
    ei = edge_index.astype(jnp.int32)
    pad = EPAD - E
    srcb = jnp.concatenate([ei[0], jnp.zeros((pad,), jnp.int32)])
    srcb = srcb.reshape(TCH, G, BLK)
    dstb = jnp.concatenate([ei[1], jnp.full((pad,), N, jnp.int32)])
    dstb = dstb.reshape(TCH, G, BLK)
    edgb = jnp.stack([srcb, dstb], axis=1)
    ones_t = jnp.ones((N, H), jnp.float32)
    zrow = jnp.zeros((RPT, H), jnp.float32)

    degp = _seg_sum_sc(ones_t, edgb, zrow, with_gather=False)
    dinv, h1p = _tc_pre(x, W1, degp)

    s1 = _seg_sum_sc(h1p, edgb, zrow, with_gather=True)
    h2p = _tc_mid(s1, h1p, dinv, b1, g1, be1, W2)

    s2 = _seg_sum_sc(h2p, edgb, zrow, with_gather=True)
    z2p = _tc_mid(s2, h2p, dinv, b2, g2, be2, None)

    s3 = _seg_sum_sc(z2p, edgb, zrow, with_gather=True)
    out = _tc_post(s3, z2p, dinv, W3, b3, g3, be3, x,
                   fw1, fb1, fw2, fb2, fw3, fb3, fw4, fb4)
    return out.reshape((1,))


# speedup vs baseline: 1.4202x; 1.2732x over previous
import functools

import jax
import jax.numpy as jnp
from jax import lax
from jax.experimental import pallas as pl
from jax.experimental.pallas import tpu as pltpu
from jax.experimental.pallas import tpu_sc as plsc

N = 10000
E = 320000
D = 128
H = 16

NC = 2
NS = 16
NW = NC * NS
BLK = 128
G = 8
SG = G * BLK
TCH = 320
EPAD = TCH * SG
K0 = 10
K1 = 20 - K0
RPT = 632
NPAD = NS * RPT


def _seg_sum_sc(table, edgb, zrow, with_gather):
    mesh = plsc.VectorSubcoreMesh(core_axis_name="c", subcore_axis_name="s")
    RLAST = N - (NS - 1) * RPT

    @functools.partial(
        pl.kernel,
        out_type=jax.ShapeDtypeStruct((NC, NPAD, H), jnp.float32),
        mesh=mesh,
        scratch_types=[
            pltpu.VMEM_SHARED((NPAD, H), jnp.float32),
            pltpu.VMEM_SHARED((N, H), jnp.float32),
            pltpu.VMEM((2, G, BLK), jnp.int32),
            pltpu.VMEM((2, G, BLK), jnp.int32),
            pltpu.VMEM((SG, H), jnp.float32),
            pltpu.VMEM((SG, H), jnp.float32),
            pltpu.VMEM((RPT, H), jnp.float32),
            pltpu.SemaphoreType.DMA,
            pltpu.SemaphoreType.DMA,
            pltpu.SemaphoreType.DMA,
            pltpu.SemaphoreType.DMA,
        ],
        compiler_params=pltpu.CompilerParams(use_tc_tiling_on_sc=False),
    )
    def k(table_h, edg_h, zrow_h, out_h,
          acc, tspm, eA, eB, rA, rB, tbuf, gsA, gsB, ssA, ssB):
        c = lax.axis_index("c")
        s = lax.axis_index("s")
        base = jnp.where(c == 0, s * K0, NS * K0 + s * K1)
        npair = jnp.where(c == 0, K0 // 2, K1 // 2)

        def stage(ebuf, sup):
            pltpu.sync_copy(edg_h.at[base + sup], ebuf)

        def fire_g(ebuf, rbuf, sem):
            for j in range(G):
                pltpu.async_copy(tspm.at[ebuf.at[0, j]],
                                 rbuf.at[pl.ds(j * BLK, BLK)], sem)

        def drain_g(rbuf, sem):
            pltpu.make_async_copy(table_h.at[pl.ds(0, SG)], rbuf, sem).wait()

        def fire_s(ebuf, rbuf, sem):
            for j in range(G):
                pltpu.async_copy(rbuf.at[pl.ds(j * BLK, BLK)],
                                 acc.at[ebuf.at[1, j]], sem, add=True)

        def drain_s(rbuf, sem):
            pltpu.make_async_copy(rbuf, acc.at[pl.ds(0, SG)], sem).wait()

        stage(eA, 0)
        if with_gather:
            @pl.when(s < NS - 1)
            def _():
                pltpu.sync_copy(table_h.at[pl.ds(s * RPT, RPT)], tbuf)
                pltpu.sync_copy(tbuf, tspm.at[pl.ds(s * RPT, RPT)])

            @pl.when(s == NS - 1)
            def _():
                pltpu.sync_copy(table_h.at[pl.ds((NS - 1) * RPT, RLAST)],
                                tbuf.at[pl.ds(0, RLAST)])
                pltpu.sync_copy(tbuf.at[pl.ds(0, RLAST)],
                                tspm.at[pl.ds((NS - 1) * RPT, RLAST)])
        else:
            pltpu.sync_copy(table_h.at[pl.ds(0, SG)], rA)
        pltpu.sync_copy(zrow_h, tbuf)
        pltpu.sync_copy(tbuf, acc.at[pl.ds(s * RPT, RPT)])
        plsc.subcore_barrier()
        if with_gather:
            fire_g(eA, rA, gsA)
        else:
            fire_s(eA, rA, ssA)

        def body(i, carry):
            i2 = 2 * i
            if with_gather:
                stage(eB, i2 + 1)
                fire_g(eB, rB, gsB)
                drain_g(rA, gsA)
                fire_s(eA, rA, ssA)
                drain_s(rA, ssA)

                @pl.when(i < npair - 1)
                def _():
                    stage(eA, i2 + 2)
                    fire_g(eA, rA, gsA)

                drain_g(rB, gsB)
                fire_s(eB, rB, ssB)
                drain_s(rB, ssB)
            else:
                stage(eB, i2 + 1)
                fire_s(eB, rA, ssB)
                drain_s(rA, ssA)

                @pl.when(i < npair - 1)
                def _():
                    stage(eA, i2 + 2)
                    fire_s(eA, rA, ssA)

                drain_s(rA, ssB)
            return carry

        lax.fori_loop(0, npair, body, 0)
        plsc.subcore_barrier()

        pltpu.sync_copy(acc.at[pl.ds(s * RPT, RPT)], tbuf)
        pltpu.sync_copy(tbuf, out_h.at[c, pl.ds(s * RPT, RPT)])

    return k(table, edgb, zrow)


def _bn_gelu(y, g, be):
    m = jnp.mean(y, axis=0, keepdims=True)
    v = jnp.mean((y - m) * (y - m), axis=0, keepdims=True)
    return jax.nn.gelu((y - m) / jnp.sqrt(v + 1e-5) * g + be)


def _tc_pre(x, W1, degp):
    def k(x_r, w_r, dg_r, dinv_r, h1p_r):
        deg = dg_r[0, :N, :] + dg_r[1, :N, :] + 1.0
        dinv = lax.rsqrt(deg)
        h1 = jnp.dot(x_r[...], w_r[...], preferred_element_type=jnp.float32, precision=lax.Precision.HIGHEST)
        dinv_r[...] = dinv
        h1p_r[...] = h1 * dinv

    return pl.pallas_call(
        k,
        out_shape=(jax.ShapeDtypeStruct((N, H), jnp.float32),
                   jax.ShapeDtypeStruct((N, H), jnp.float32)),
    )(x, W1, degp)


def _tc_mid(sp, hp, dinv, b, g, be, W):
    def k(sp_r, hp_r, dinv_r, b_r, g_r, be_r, *rest):
        w_r, out_r = (rest[0], rest[1]) if len(rest) == 2 else (None, rest[0])
        y = dinv_r[...] * (sp_r[0, :N, :] + sp_r[1, :N, :] + hp_r[...]) + b_r[...]
        z = _bn_gelu(y, g_r[...], be_r[...])
        if w_r is not None:
            z = jnp.dot(z, w_r[...], preferred_element_type=jnp.float32, precision=lax.Precision.HIGHEST)
        out_r[...] = z * dinv_r[...]

    args = [sp, hp, dinv, b.reshape(1, H), g.reshape(1, H), be.reshape(1, H)]
    if W is not None:
        args.append(W)
    return pl.pallas_call(
        k,
        out_shape=jax.ShapeDtypeStruct((N, H), jnp.float32),
    )(*args)


def _tc_post(s3, z2p, dinv, W3, b3, g3, be3, x,
             fw1, fb1, fw2, fb2, fw3, fb3, fw4, fb4):
    def k(s3_r, z2p_r, dinv_r, w3_r, b3_r, g3_r, be3_r, x_r,
          f1_r, c1_r, f2_r, c2_r, f3_r, c3_r, f4_r, c4_r, out_r):
        t = dinv_r[...] * (s3_r[0, :N, :] + s3_r[1, :N, :] + z2p_r[...])
        h3 = jnp.dot(t, w3_r[...], preferred_element_type=jnp.float32, precision=lax.Precision.HIGHEST) + b3_r[...]
        z3 = _bn_gelu(h3, g3_r[...], be3_r[...])
        hh = x_r[...] + z3
        v = jnp.mean(hh, axis=0, keepdims=True)
        v = jax.nn.gelu(jnp.dot(v, f1_r[...], preferred_element_type=jnp.float32, precision=lax.Precision.HIGHEST) + c1_r[...])
        v = jax.nn.gelu(jnp.dot(v, f2_r[...], preferred_element_type=jnp.float32, precision=lax.Precision.HIGHEST) + c2_r[...])
        v = jax.nn.gelu(jnp.dot(v, f3_r[...], preferred_element_type=jnp.float32, precision=lax.Precision.HIGHEST) + c3_r[...])
        out_r[...] = jnp.dot(v, f4_r[...], preferred_element_type=jnp.float32, precision=lax.Precision.HIGHEST) + c4_r[...]

    return pl.pallas_call(
        k,
        out_shape=jax.ShapeDtypeStruct((1, 1), jnp.float32),
    )(s3, z2p, dinv, W3, b3.reshape(1, D), g3.reshape(1, D), be3.reshape(1, D),
      x, fw1, fb1.reshape(1, 128), fw2, fb2.reshape(1, 64),
      fw3, fb3.reshape(1, 32), fw4, fb4.reshape(1, 1))


def kernel(x, edge_index, W1, b1, g1, be1, W2, b2, g2, be2, W3, b3, g3, be3,
           fw1, fb1, fw2, fb2, fw3, fb3, fw4, fb4):
    ei = edge_index.astype(jnp.int32)
    pad = EPAD - E
    srcb = jnp.concatenate([ei[0], jnp.zeros((pad,), jnp.int32)])
    srcb = srcb.reshape(TCH, G, BLK)
    dstb = jnp.concatenate([ei[1], jnp.full((pad,), N, jnp.int32)])
    dstb = dstb.reshape(TCH, G, BLK)
    edgb = jnp.stack([srcb, dstb], axis=1)
    ones_t = jnp.ones((N, H), jnp.float32)
    zrow = jnp.zeros((RPT, H), jnp.float32)

    degp = _seg_sum_sc(ones_t, edgb, zrow, with_gather=False)
    dinv, h1p = _tc_pre(x, W1, degp)

    s1 = _seg_sum_sc(h1p, edgb, zrow, with_gather=True)
    h2p = _tc_mid(s1, h1p, dinv, b1, g1, be1, W2)

    s2 = _seg_sum_sc(h2p, edgb, zrow, with_gather=True)
    z2p = _tc_mid(s2, h2p, dinv, b2, g2, be2, None)

    s3 = _seg_sum_sc(z2p, edgb, zrow, with_gather=True)
    out = _tc_post(s3, z2p, dinv, W3, b3, g3, be3, x,
                   fw1, fb1, fw2, fb2, fw3, fb3, fw4, fb4)
    return out.reshape((1,))
